# baseline (speedup 1.0000x reference)
import jax
import jax.numpy as jnp
from jax import lax
from jax.experimental import pallas as pl
from jax.experimental.pallas import tpu as pltpu
from jax.experimental.pallas import tpu_sc as plsc

_NC = 2
_NS = 16
_NW = _NC * _NS


def kernel(token_indices, lookup):
    if token_indices.ndim == 1:
        token_indices = token_indices[None, :]
    B, S = token_indices.shape
    V, D = lookup.shape
    N = B * S
    PR = 128 // D
    b_per_w = N // _NW
    C = 320
    CL = C // PR
    n_chunks = b_per_w // C
    assert b_per_w % C == 0 and n_chunks % 2 == 0 and n_chunks >= 4

    idx = token_indices.reshape(N).astype(jnp.int32)
    tabv = lookup.reshape(V // PR, D * PR)
    mesh = plsc.VectorSubcoreMesh(core_axis_name="core", subcore_axis_name="subcore")

    @pl.kernel(
        out_type=jax.ShapeDtypeStruct((N // PR, D * PR), lookup.dtype),
        mesh=mesh,
        scratch_types=(
            [pltpu.VMEM((b_per_w,), jnp.int32),
             pltpu.VMEM((C,), jnp.int32),
             pltpu.VMEM((C,), jnp.int32),
             pltpu.VMEM((2, C, D * PR), lookup.dtype),
             pltpu.VMEM((2, CL, D * PR), lookup.dtype)]
            + [pltpu.SemaphoreType.DMA] * 5
        ),
    )
    def gather_kernel(table_hbm, idx_hbm, out_hbm, tok_v, sr0_v, sr1_v, gbuf, obuf,
                      *sems):
        tsem = sems[0]
        gsem = sems[1:3]
        osem = sems[3:5]
        sr_v = (sr0_v, sr1_v)
        wid = lax.axis_index("subcore") * _NC + lax.axis_index("core")
        base = wid * b_per_w

        def g_copy(b):
            return pltpu.make_async_copy(
                table_hbm.at[sr_v[b]], gbuf.at[b], gsem[b])

        def o_copy(g, b):
            off = pl.multiple_of((base + g * C) // PR, 8)
            return pltpu.make_async_copy(
                obuf.at[b], out_hbm.at[pl.ds(off, CL)], osem[b])

        def compute_sr(g, b):
            @pl.loop(0, C, step=16)
            def _(j):
                sr_v[b][pl.ds(j, 16)] = tok_v[pl.ds(g * C + j, 16)] >> 2

        pltpu.async_copy(idx_hbm.at[pl.ds(base, b_per_w)], tok_v, tsem).wait()
        compute_sr(0, 0)
        g_copy(0).start()

        @pl.loop(0, n_chunks, step=2)
        def _(gi):
            for b in range(2):
                g = gi + b
                g_copy(b).wait()

                @pl.when(g + 1 < n_chunks)
                def _():
                    compute_sr(g + 1, 1 - b)
                    g_copy(1 - b).start()

                @pl.when(g >= 2)
                def _():
                    o_copy(g - 2, b).wait()

                @pl.loop(0, C, step=16)
                def _(j):
                    tv = tok_v[pl.ds(g * C + j, 16)]
                    for l in range(16):
                        col = (tv[l] & (PR - 1)) * D
                        for h in range(D // 16):
                            obuf[b, j // PR + l // PR,
                                 pl.ds((l % PR) * D + 16 * h, 16)] = (
                                gbuf[b, j + l, pl.ds(col + 16 * h, 16)])

                o_copy(g, b).start()

        o_copy(n_chunks - 2, 0).wait()
        o_copy(n_chunks - 1, 1).wait()

    lin = gather_kernel(tabv, idx)
    return lin.reshape(B, S, D)
